# baseline (speedup 1.0000x reference)
import numpy as np

import jax
import jax.numpy as jnp
from jax.experimental import pallas as pl
from jax.experimental.pallas import tpu as pltpu


def _structure_matrix(T, C, F):
    r = np.arange(T * F * C)
    t = r // (F * C)
    f = (r // C) % F
    oh = (f[:, None] == np.arange(F)[None, :]).astype(np.float32)
    m_prev = (t >= 2).astype(np.float32)[:, None]
    m_cur = (t >= 1).astype(np.float32)[:, None]
    m_next = (t <= T - 2).astype(np.float32)[:, None]
    return np.concatenate(
        [oh * m_prev, oh * m_cur, oh * m_next, oh], axis=1)


def _make_body(T, C, F, eps):
    def body(x_ref, oh2_ref, wc_ref, p_ref, o_ref):
        P = p_ref[...]
        bc = P[0:1]
        gamma = P[1:2]
        beta = P[2:3]
        rmean = P[3:4]
        rvar = P[4:5]

        inv_std = 1.0 / jnp.sqrt(rvar + eps)
        scale = gamma * inv_std
        shift = beta - rmean * scale

        ones_f = jnp.ones((1, F), jnp.float32)
        s0 = jnp.dot(ones_f, wc_ref[0], preferred_element_type=jnp.float32)
        s1 = jnp.dot(ones_f, wc_ref[1], preferred_element_type=jnp.float32)
        s2 = jnp.dot(ones_f, wc_ref[2], preferred_element_type=jnp.float32)

        val = 0.0 - jnp.log(jnp.float32(C) * jnp.exp(jnp.float32(0.0)))

        vs = val * scale
        v_row = jnp.concatenate(
            [vs * s0, vs * s1, vs * s2, scale * bc + shift], axis=1)
        dn = (((1,), (1,)), ((), ()))
        a2 = jax.lax.dot_general(oh2_ref[...], v_row, dn,
                                 preferred_element_type=jnp.float32)

        o_ref[...] = x_ref[...] + a2

    return body


@jax.jit
def kernel(features, adjs, w1, b1, w2, b2, wc, bc,
           gamma, beta, rmean, rvar):
    del adjs, w1, b1, w2, b2
    eps = 1e-5
    B, T, C, F = features.shape
    R = T * F * C

    x2 = jnp.transpose(features, (1, 3, 2, 0)).reshape(R, B)
    oh2 = jnp.asarray(_structure_matrix(T, C, F))

    P = jnp.concatenate(
        [bc[None], gamma[None], beta[None], rmean[None], rvar[None]],
        axis=0)

    LB = 128 if B % 128 == 0 else B
    grid = (B // LB,)

    out2 = pl.pallas_call(
        _make_body(T, C, F, eps),
        out_shape=jax.ShapeDtypeStruct((R, B), jnp.float32),
        grid=grid,
        in_specs=[
            pl.BlockSpec((R, LB), lambda g: (0, g)),
            pl.BlockSpec((R, 4 * F), lambda g: (0, 0)),
            pl.BlockSpec((3, F, F), lambda g: (0, 0, 0)),
            pl.BlockSpec((5, F), lambda g: (0, 0)),
        ],
        out_specs=pl.BlockSpec((R, LB), lambda g: (0, g)),
        compiler_params=pltpu.CompilerParams(
            dimension_semantics=("parallel",)),
    )(x2, oh2, wc, P)

    return out2.reshape(T, F, C, B).transpose(3, 0, 2, 1)


# speedup vs baseline: 1.2111x; 1.2111x over previous
import numpy as np

import jax
import jax.numpy as jnp
from jax.experimental import pallas as pl
from jax.experimental.pallas import tpu as pltpu


def _structure_matrix(T, C, F):
    r = np.arange(T * F * C)
    t = r // (F * C)
    f = (r // C) % F
    oh = (f[:, None] == np.arange(F)[None, :]).astype(np.float32)
    m_prev = (t >= 2).astype(np.float32)[:, None]
    m_cur = (t >= 1).astype(np.float32)[:, None]
    m_next = (t <= T - 2).astype(np.float32)[:, None]
    return np.concatenate(
        [oh * m_prev, oh * m_cur, oh * m_next, oh], axis=1)


def _make_body(T, C, F, eps):
    def body(x_ref, oh2_ref, wc_ref, p_ref, o_ref):
        P = p_ref[...]
        bc = P[0:1]
        gamma = P[1:2]
        beta = P[2:3]
        rmean = P[3:4]
        rvar = P[4:5]

        inv_std = 1.0 / jnp.sqrt(rvar + eps)
        scale = gamma * inv_std
        shift = beta - rmean * scale

        ones_f = jnp.ones((1, F), jnp.float32)
        s0 = jnp.dot(ones_f, wc_ref[0], preferred_element_type=jnp.float32)
        s1 = jnp.dot(ones_f, wc_ref[1], preferred_element_type=jnp.float32)
        s2 = jnp.dot(ones_f, wc_ref[2], preferred_element_type=jnp.float32)

        val = 0.0 - jnp.log(jnp.float32(C) * jnp.exp(jnp.float32(0.0)))

        vs = val * scale
        v_row = jnp.concatenate(
            [vs * s0, vs * s1, vs * s2, scale * bc + shift], axis=1)
        dn = (((1,), (1,)), ((), ()))
        a2 = jax.lax.dot_general(oh2_ref[...], v_row, dn,
                                 preferred_element_type=jnp.float32)

        o_ref[...] = x_ref[...] + a2

    return body


@jax.jit
def kernel(features, adjs, w1, b1, w2, b2, wc, bc,
           gamma, beta, rmean, rvar):
    del adjs, w1, b1, w2, b2
    eps = 1e-5
    B, T, C, F = features.shape
    R = T * F * C

    x2 = jnp.transpose(features, (1, 3, 2, 0)).reshape(R, B)
    oh2 = jnp.asarray(_structure_matrix(T, C, F))

    P = jnp.concatenate(
        [bc[None], gamma[None], beta[None], rmean[None], rvar[None]],
        axis=0)

    LB = B // 2 if B % 256 == 0 else B
    grid = (B // LB,)

    out2 = pl.pallas_call(
        _make_body(T, C, F, eps),
        out_shape=jax.ShapeDtypeStruct((R, B), jnp.float32),
        grid=grid,
        in_specs=[
            pl.BlockSpec((R, LB), lambda g: (0, g)),
            pl.BlockSpec((R, 4 * F), lambda g: (0, 0)),
            pl.BlockSpec((3, F, F), lambda g: (0, 0, 0)),
            pl.BlockSpec((5, F), lambda g: (0, 0)),
        ],
        out_specs=pl.BlockSpec((R, LB), lambda g: (0, g)),
        compiler_params=pltpu.CompilerParams(
            dimension_semantics=("parallel",)),
    )(x2, oh2, wc, P)

    return out2.reshape(T, F, C, B).transpose(3, 0, 2, 1)
